# baseline (speedup 1.0000x reference)
import functools

import jax
import jax.numpy as jnp
from jax import lax
from jax.experimental import pallas as pl
from jax.experimental.pallas import tpu as pltpu
from jax.experimental.pallas import tpu_sc as plsc

BATCH = 16384
VOCAB = 1000000
DEPTH = 64
NC = 2
NS = 16
NW = NC * NS
BPW = BATCH // NW
G = 128
NG = BPW // G

_mesh = plsc.VectorSubcoreMesh(core_axis_name="c", subcore_axis_name="s")


@functools.partial(
    pl.kernel,
    mesh=_mesh,
    out_type=jax.ShapeDtypeStruct((BATCH, 2 * DEPTH), jnp.float32),
    scratch_types=[
        pltpu.VMEM((NG, G), jnp.int32),
        pltpu.VMEM((BPW, 2 * DEPTH), jnp.float32),
        pltpu.SemaphoreType.DMA,
    ],
    compiler_params=pltpu.CompilerParams(skip_device_barrier=True),
)
def _gather_kernel(idx_hbm, table_hbm, out_hbm, idx_v, stage_v, sem):
    wid = lax.axis_index("s") * NC + lax.axis_index("c")
    pltpu.sync_copy(idx_hbm.at[wid], idx_v)
    copies = []
    for j in range(NG):
        copies.append(
            pltpu.async_copy(
                table_hbm.at[idx_v.at[j]], stage_v.at[pl.ds(j * G, G)], sem))
    for cp in copies:
        cp.wait()
    pltpu.sync_copy(stage_v, out_hbm.at[pl.ds(wid * BPW, BPW)])


def kernel(x, embedding):
    flat = jnp.reshape(x, (BATCH,))
    idx = jnp.reshape(flat >> 1, (NW, NG, G))
    packed = jnp.reshape(embedding, (VOCAB // 2, 2 * DEPTH))
    pairs = _gather_kernel(idx, packed)
    odd = (flat & 1)[:, None] == 1
    return jnp.where(odd, pairs[:, DEPTH:], pairs[:, :DEPTH])


# speedup vs baseline: 1.3873x; 1.3873x over previous
import functools

import jax
import jax.numpy as jnp
from jax import lax
from jax.experimental import pallas as pl
from jax.experimental.pallas import tpu as pltpu
from jax.experimental.pallas import tpu_sc as plsc

BATCH = 16384
VOCAB = 1000000
DEPTH = 64
NC = 2
NS = 16
NW = NC * NS
BPW = BATCH // NW
NSR = 320
NDR = BPW - NSR

_mesh = plsc.VectorSubcoreMesh(core_axis_name="c", subcore_axis_name="s")


@functools.partial(
    pl.kernel,
    mesh=_mesh,
    out_type=jax.ShapeDtypeStruct((BATCH, DEPTH), jnp.float32),
    scratch_types=[
        pltpu.VMEM((BPW,), jnp.int32),
        pltpu.VMEM((NSR, DEPTH), jnp.float32),
        pltpu.SemaphoreType.DMA,
        pltpu.SemaphoreType.DMA,
    ],
)
def _gather_kernel(idx_hbm, table_hbm, out_hbm, idx_vm, stage_v, sem_s, sem_d):
    wid = lax.axis_index("s") * NC + lax.axis_index("c")
    pltpu.sync_copy(idx_hbm.at[wid], idx_vm)
    base = wid * BPW

    def stream_body(g, carry):
        b16 = g * 16
        v = idx_vm[pl.ds(b16, 16)]
        for l in range(16):
            pltpu.async_copy(
                table_hbm.at[v[l]], stage_v.at[b16 + l], sem_s)
        return carry

    def direct_body(g, carry):
        b16 = g * 16
        v = idx_vm[pl.ds(b16, 16)]
        for l in range(16):
            pltpu.async_copy(
                table_hbm.at[v[l]], out_hbm.at[base + b16 + l], sem_d)
        return carry

    lax.fori_loop(0, NSR // 16, stream_body, 0)
    lax.fori_loop(NSR // 16, BPW // 16, direct_body, 0)
    pltpu.make_async_copy(
        table_hbm.at[pl.ds(0, NDR)],
        out_hbm.at[pl.ds(base + NSR, NDR)], sem_d).wait()
    pltpu.make_async_copy(
        table_hbm.at[pl.ds(0, NSR)], stage_v, sem_s).wait()
    pltpu.sync_copy(stage_v, out_hbm.at[pl.ds(base, NSR)])


def kernel(x, embedding):
    idx = jnp.reshape(x, (NW, BPW))
    return _gather_kernel(idx, embedding)


# speedup vs baseline: 1.7357x; 1.2511x over previous
import functools

import jax
import jax.numpy as jnp
from jax import lax
from jax.experimental import pallas as pl
from jax.experimental.pallas import tpu as pltpu
from jax.experimental.pallas import tpu_sc as plsc

BATCH = 16384
VOCAB = 1000000
DEPTH = 64
NC = 2
NS = 16
NW = NC * NS
BPW = BATCH // NW

_mesh = plsc.VectorSubcoreMesh(core_axis_name="c", subcore_axis_name="s")


@functools.partial(
    pl.kernel,
    mesh=_mesh,
    out_type=jax.ShapeDtypeStruct((BATCH, DEPTH), jnp.float32),
    scratch_types=[
        pltpu.VMEM((BPW,), jnp.int32),
        pltpu.VMEM((BPW, DEPTH), jnp.float32),
        pltpu.SemaphoreType.DMA,
    ],
)
def _gather_kernel(idx_hbm, table_hbm, out_hbm, idx_vm, stage_v, sem):
    wid = lax.axis_index("s") * NC + lax.axis_index("c")
    pltpu.sync_copy(idx_hbm.at[wid], idx_vm)

    def body(g, carry):
        base = g * 16
        v = idx_vm[pl.ds(base, 16)]
        for l in range(16):
            pltpu.async_copy(table_hbm.at[v[l]], stage_v.at[base + l], sem)
        return carry

    lax.fori_loop(0, BPW // 16, body, 0)
    pltpu.make_async_copy(table_hbm.at[pl.ds(0, BPW)], stage_v, sem).wait()
    pltpu.sync_copy(stage_v, out_hbm.at[pl.ds(wid * BPW, BPW)])


def kernel(x, embedding):
    idx = jnp.reshape(x, (NW, BPW))
    return _gather_kernel(idx, embedding)
